# baseline (speedup 1.0000x reference)
import jax
import jax.numpy as jnp
from jax.experimental import pallas as pl
from jax.experimental.pallas import tpu as pltpu

_N_TILES = 4


def _body(x_hbm, out_hbm, sem):
    for i in range(32):
        b, t = i // _N_TILES, i % _N_TILES
        pltpu.make_async_copy(x_hbm.at[b, t], out_hbm.at[b, t], sem).start()
    for i in range(32):
        b, t = i // _N_TILES, i % _N_TILES
        pltpu.make_async_copy(x_hbm.at[b, t], out_hbm.at[b, t], sem).wait()


def kernel(x, aspect_ratio, global_positional_embedding, local_positional_embedding, gate):
    bsz, n_tiles, num_tokens, embed_dim = x.shape
    return pl.pallas_call(
        _body,
        in_specs=[pl.BlockSpec(memory_space=pltpu.MemorySpace.HBM)],
        out_specs=pl.BlockSpec(memory_space=pltpu.MemorySpace.HBM),
        out_shape=jax.ShapeDtypeStruct((bsz, n_tiles, num_tokens, embed_dim), x.dtype),
        scratch_shapes=[pltpu.SemaphoreType.DMA],
    )(x)


# speedup vs baseline: 10.9172x; 10.9172x over previous
import jax
import jax.numpy as jnp
from jax.experimental import pallas as pl
from jax.experimental.pallas import tpu as pltpu

_N_TILES = 4
_N_SLICES = 32
_K = 4


def _body(ar_ref, gate_ref, x_hbm, local_hbm, gpe_hbm, out_hbm,
          ibuf, obuf, lbuf, gchunk, isem, osem, lsem, gsem):
    tg = jnp.tanh(gate_ref[0])
    a = 1.0 - tg
    gate_on = tg != 0.0

    def in_copy(k, slot):
        b = k // _N_TILES
        t = k % _N_TILES
        return pltpu.make_async_copy(
            x_hbm.at[b, t], ibuf.at[slot], isem.at[slot])

    def out_copy(k, slot):
        b = k // _N_TILES
        t = k % _N_TILES
        return pltpu.make_async_copy(
            obuf.at[slot], out_hbm.at[b, t], osem.at[slot])

    pltpu.make_async_copy(local_hbm, lbuf, lsem).start()

    for k in range(_K - 1):
        in_copy(k, k).start()

    pltpu.make_async_copy(local_hbm, lbuf, lsem).wait()

    def step(k, carry):
        slot = k % _K
        b = k // _N_TILES
        t = k % _N_TILES
        h = ar_ref[b, 0]
        w = ar_ref[b, 1]
        w_safe = jnp.maximum(w, 1)
        row = t // w_safe
        col = t % w_safe
        valid = t < h * w
        fetch = jnp.logical_and(valid, gate_on)

        in_copy(k, slot).wait()

        @pl.when(k >= _K)
        def _free_out_slot():
            out_copy(k - _K, slot).wait()

        @pl.when(fetch)
        def _fetch_global():
            g = pltpu.make_async_copy(gpe_hbm.at[row, col], gchunk, gsem)
            g.start()
            g.wait()

        obuf[slot] = ibuf[slot] + lbuf[...] * a

        @pl.when(fetch)
        def _add_global():
            obuf[slot] += gchunk[...] * tg

        out_copy(k, slot).start()

        @pl.when(k + _K - 1 < _N_SLICES)
        def _prefetch():
            in_copy(k + _K - 1, (k + _K - 1) % _K).start()

        return carry

    jax.lax.fori_loop(0, _N_SLICES, step, 0)

    for k in range(_N_SLICES - _K, _N_SLICES):
        out_copy(k, k % _K).wait()


def kernel(x, aspect_ratio, global_positional_embedding, local_positional_embedding, gate):
    bsz, n_tiles, num_tokens, embed_dim = x.shape
    ar = aspect_ratio.astype(jnp.int32)

    return pl.pallas_call(
        _body,
        in_specs=[
            pl.BlockSpec(memory_space=pltpu.SMEM),
            pl.BlockSpec(memory_space=pltpu.SMEM),
            pl.BlockSpec(memory_space=pltpu.MemorySpace.HBM),
            pl.BlockSpec(memory_space=pltpu.MemorySpace.HBM),
            pl.BlockSpec(memory_space=pltpu.MemorySpace.HBM),
        ],
        out_specs=pl.BlockSpec(memory_space=pltpu.MemorySpace.HBM),
        out_shape=jax.ShapeDtypeStruct((bsz, n_tiles, num_tokens, embed_dim), x.dtype),
        scratch_shapes=[
            pltpu.VMEM((_K, num_tokens, embed_dim), jnp.float32),
            pltpu.VMEM((_K, num_tokens, embed_dim), jnp.float32),
            pltpu.VMEM((num_tokens, embed_dim), jnp.float32),
            pltpu.VMEM((num_tokens, embed_dim), jnp.float32),
            pltpu.SemaphoreType.DMA((_K,)),
            pltpu.SemaphoreType.DMA((_K,)),
            pltpu.SemaphoreType.DMA,
            pltpu.SemaphoreType.DMA,
        ],
    )(ar, gate, x, local_positional_embedding, global_positional_embedding)
